# Initial kernel scaffold; baseline (speedup 1.0000x reference)
import jax
import jax.numpy as jnp
from jax.experimental import pallas as pl


def kernel(x, edge_index, W_init0, b_init0, W_init1, b_init1, W_pos0, b_pos0, W_pos1, b_pos1, W_local, b_local, W_global, b_global, W_final, b_final):
    raise NotImplementedError("write your pallas kernel here")


# speedup vs baseline: 1.0660x; 1.0660x over previous
import functools

import jax
import jax.numpy as jnp
from jax import lax
from jax.experimental import pallas as pl
from jax.experimental.pallas import tpu as pltpu
from jax.experimental.pallas import tpu_sc as plsc

N = 10000
NP = 10240
F = 128
E = 320000
CPS = 4
EK = 16000
NCHUNK = E // EK
GRP = 16

BLK = 2048


def _prologue_body(x_ref, w0_ref, b0_ref, w1_ref, b1_ref, wp0_ref, bp0_ref,
                   wp1_ref, bp1_ref, wp_ref, bl_ref, wg_ref, bg_ref, wh_ref,
                   g_ref, c_ref, d_ref):
    dot = functools.partial(jnp.dot, preferred_element_type=jnp.float32)
    xb = x_ref[...]
    h0 = jnp.maximum(dot(w0_ref[...], xb) + b0_ref[...], 0.0)
    h = dot(w1_ref[...], h0) + b1_ref[...]
    t = jnp.maximum(dot(wp0_ref[...], h) + bp0_ref[...], 0.0)
    pos = dot(wp1_ref[...], t) + bp1_ref[...]
    p = dot(wp_ref[...], pos)
    c = p + bl_ref[...]
    d_ref[...] = bg_ref[...] - dot(wg_ref[...], p)
    c_ref[...] = c
    g_ref[...] = dot(wh_ref[...], h) + c


def _layer_body(m_ref, c_ref, d_ref, wg_ref, wh_ref, g_ref):
    dot = functools.partial(jnp.dot, preferred_element_type=jnp.float32)
    h = jnp.maximum(dot(wg_ref[...], m_ref[...]) + d_ref[...], 0.0)
    g_ref[...] = dot(wh_ref[...], h) + c_ref[...]


def _final_body(m_ref, d_ref, wg_ref, wf_ref, bf_ref, o_ref):
    dot = functools.partial(jnp.dot, preferred_element_type=jnp.float32)
    h = jnp.maximum(dot(wg_ref[...], m_ref[...]) + d_ref[...], 0.0)
    o_ref[...] = dot(wf_ref[...], h) + bf_ref[...]


def _col_spec():
    return pl.BlockSpec((F, BLK), lambda j: (0, j))


def _fix(shape):
    return pl.BlockSpec(shape, lambda j: (0, 0))


_GRID = (NP // BLK,)
_ACT = jax.ShapeDtypeStruct((F, NP), jnp.float32)

_prologue = pl.pallas_call(
    _prologue_body,
    grid=_GRID,
    in_specs=[
        _col_spec(),
        _fix((F, F)), _fix((F, 1)),
        _fix((F, F)), _fix((F, 1)),
        _fix((F, F)), _fix((F, 1)),
        _fix((3, F)), _fix((3, 1)),
        _fix((F, 3)),
        _fix((F, 1)),
        _fix((F, F)), _fix((F, 1)),
        _fix((F, F)),
    ],
    out_specs=[_col_spec(), _col_spec(), _col_spec()],
    out_shape=[_ACT, _ACT, _ACT],
)

_layer = pl.pallas_call(
    _layer_body,
    grid=_GRID,
    in_specs=[_col_spec(), _col_spec(), _col_spec(), _fix((F, F)), _fix((F, F))],
    out_specs=_col_spec(),
    out_shape=_ACT,
)

_final = pl.pallas_call(
    _final_body,
    grid=_GRID,
    in_specs=[_col_spec(), _col_spec(), _fix((F, F)), _fix((F, F)), _fix((F, 1))],
    out_specs=_col_spec(),
    out_shape=_ACT,
)


_MESH = plsc.VectorSubcoreMesh(core_axis_name="c", subcore_axis_name="s")


@functools.partial(
    pl.kernel,
    out_type=jax.ShapeDtypeStruct((F, NP), jnp.float32),
    mesh=_MESH,
    compiler_params=pltpu.CompilerParams(needs_layout_passes=False),
    scratch_types=[
        pltpu.VMEM((CPS * NP,), jnp.float32),
        pltpu.VMEM((CPS * NP,), jnp.float32),
        pltpu.VMEM((EK,), jnp.int32),
        pltpu.VMEM((EK,), jnp.int32),
    ],
)
def _segmax(g_hbm, src_hbm, dst_hbm, out_hbm, gt_v, acc_v, srcb_v, dstb_v):
    cid = lax.axis_index("c")
    sid = lax.axis_index("s")
    wid = sid * 2 + cid
    c0 = wid * CPS
    for k in range(CPS):
        pltpu.sync_copy(g_hbm.at[c0 + k], gt_v.at[pl.ds(k * NP, NP)])
        pltpu.sync_copy(g_hbm.at[c0 + k], acc_v.at[pl.ds(k * NP, NP)])

    def chunk_body(ci, carry):
        base = ci * EK
        pltpu.sync_copy(src_hbm.at[pl.ds(base, EK)], srcb_v)
        pltpu.sync_copy(dst_hbm.at[pl.ds(base, EK)], dstb_v)

        def grp_body(gi, carry2):
            off = gi * GRP
            srcv = srcb_v[pl.ds(off, GRP)]
            dstv = dstb_v[pl.ds(off, GRP)]
            for c in range(CPS):
                coff = c * NP
                vals = plsc.load_gather(gt_v, [srcv + coff])
                didx = dstv + coff

                def rcond(act):
                    return jnp.any(act)

                def rbody(act):
                    cur = plsc.load_gather(acc_v, [didx])
                    plsc.store_scatter(acc_v, [didx],
                                       jnp.maximum(cur, vals), mask=act)
                    chk = plsc.load_gather(acc_v, [didx])
                    return act & (chk < vals)

                lax.while_loop(rcond, rbody, jnp.ones((GRP,), jnp.bool_))
            return carry2

        return lax.fori_loop(0, EK // GRP, grp_body, carry)

    lax.fori_loop(0, NCHUNK, chunk_body, 0)
    for k in range(CPS):
        pltpu.sync_copy(acc_v.at[pl.ds(k * NP, NP)], out_hbm.at[c0 + k])


def kernel(x, edge_index, W_init0, b_init0, W_init1, b_init1, W_pos0, b_pos0,
           W_pos1, b_pos1, W_local, b_local, W_global, b_global, W_final,
           b_final):
    xt = jnp.pad(x.T, ((0, 0), (0, NP - N)))
    src = edge_index[0]
    dst = edge_index[1]
    col = lambda b: b.reshape(-1, 1)

    g, cmat, dmat = _prologue(
        xt,
        W_init0.T, col(b_init0),
        W_init1.T, col(b_init1),
        W_pos0.T, col(b_pos0),
        W_pos1.T, col(b_pos1),
        W_local[F:].T,
        col(b_local),
        W_global.T, col(b_global),
        W_local[:F].T,
    )
    wgt = W_global.T
    wht = W_local[:F].T
    for i in range(2):
        m = _segmax(g, src, dst)
        g = _layer(m, cmat, dmat, wgt, wht)
    m = _segmax(g, src, dst)
    out_t = _final(m, dmat, wgt, W_final.T, col(b_final))
    return out_t[:, :N].T


# speedup vs baseline: 2.4201x; 2.2703x over previous
import functools

import jax
import jax.numpy as jnp
from jax import lax
from jax.experimental import pallas as pl
from jax.experimental.pallas import tpu as pltpu
from jax.experimental.pallas import tpu_sc as plsc

N = 10000
NP = 10240
F = 128
E = 320000
CPS = 4
EK = 8000
NCHUNK = E // EK
GRP = 16

BLK = 2048


def _prologue_body(x_ref, w0_ref, b0_ref, w1_ref, b1_ref, wp0_ref, bp0_ref,
                   wp1_ref, bp1_ref, wp_ref, bl_ref, wg_ref, bg_ref, wh_ref,
                   g_ref, c_ref, d_ref):
    dot = functools.partial(jnp.dot, preferred_element_type=jnp.float32)
    xb = x_ref[...]
    h0 = jnp.maximum(dot(w0_ref[...], xb) + b0_ref[...], 0.0)
    h = dot(w1_ref[...], h0) + b1_ref[...]
    t = jnp.maximum(dot(wp0_ref[...], h) + bp0_ref[...], 0.0)
    pos = dot(wp1_ref[...], t) + bp1_ref[...]
    p = dot(wp_ref[...], pos)
    c = p + bl_ref[...]
    d_ref[...] = bg_ref[...] - dot(wg_ref[...], p)
    c_ref[...] = c
    g_ref[...] = dot(wh_ref[...], h) + c


def _layer_body(m_ref, c_ref, d_ref, wg_ref, wh_ref, g_ref):
    dot = functools.partial(jnp.dot, preferred_element_type=jnp.float32)
    h = jnp.maximum(dot(wg_ref[...], m_ref[...]) + d_ref[...], 0.0)
    g_ref[...] = dot(wh_ref[...], h) + c_ref[...]


def _final_body(m_ref, d_ref, wg_ref, wf_ref, bf_ref, o_ref):
    dot = functools.partial(jnp.dot, preferred_element_type=jnp.float32)
    h = jnp.maximum(dot(wg_ref[...], m_ref[...]) + d_ref[...], 0.0)
    o_ref[...] = dot(wf_ref[...], h) + bf_ref[...]


def _col_spec():
    return pl.BlockSpec((F, BLK), lambda j: (0, j))


def _fix(shape):
    return pl.BlockSpec(shape, lambda j: (0, 0))


_GRID = (NP // BLK,)
_ACT = jax.ShapeDtypeStruct((F, NP), jnp.float32)

_prologue = pl.pallas_call(
    _prologue_body,
    grid=_GRID,
    in_specs=[
        _col_spec(),
        _fix((F, F)), _fix((F, 1)),
        _fix((F, F)), _fix((F, 1)),
        _fix((F, F)), _fix((F, 1)),
        _fix((3, F)), _fix((3, 1)),
        _fix((F, 3)),
        _fix((F, 1)),
        _fix((F, F)), _fix((F, 1)),
        _fix((F, F)),
    ],
    out_specs=[_col_spec(), _col_spec(), _col_spec()],
    out_shape=[_ACT, _ACT, _ACT],
)

_layer = pl.pallas_call(
    _layer_body,
    grid=_GRID,
    in_specs=[_col_spec(), _col_spec(), _col_spec(), _fix((F, F)), _fix((F, F))],
    out_specs=_col_spec(),
    out_shape=_ACT,
)

_final = pl.pallas_call(
    _final_body,
    grid=_GRID,
    in_specs=[_col_spec(), _col_spec(), _fix((F, F)), _fix((F, F)), _fix((F, 1))],
    out_specs=_col_spec(),
    out_shape=_ACT,
)


_MESH = plsc.VectorSubcoreMesh(core_axis_name="c", subcore_axis_name="s")


@functools.partial(
    pl.kernel,
    out_type=jax.ShapeDtypeStruct((F, NP), jnp.float32),
    mesh=_MESH,
    compiler_params=pltpu.CompilerParams(needs_layout_passes=False),
    scratch_types=(
        [pltpu.VMEM((NP,), jnp.float32) for _ in range(CPS)]
        + [pltpu.VMEM((NP,), jnp.float32) for _ in range(CPS)]
        + [pltpu.VMEM((EK,), jnp.int32) for _ in range(4)]
        + [pltpu.SemaphoreType.DMA for _ in range(4)]
    ),
)
def _segmax(g_hbm, src_hbm, dst_hbm, out_hbm,
            gt0, gt1, gt2, gt3, ac0, ac1, ac2, ac3,
            src0, src1, dst0, dst1, sem0, sem1, sem2, sem3):
    cid = lax.axis_index("c")
    sid = lax.axis_index("s")
    wid = sid * 2 + cid
    c0 = wid * CPS
    gts = (gt0, gt1, gt2, gt3)
    acs = (ac0, ac1, ac2, ac3)
    srcb = (src0, src1)
    dstb = (dst0, dst1)
    sems = ((sem0, sem1), (sem2, sem3))
    for k in range(CPS):
        pltpu.sync_copy(g_hbm.at[c0 + k], gts[k])
        pltpu.sync_copy(g_hbm.at[c0 + k], acs[k])

    def fetch(ci, slot):
        base = ci * EK
        a = pltpu.async_copy(src_hbm.at[pl.ds(base, EK)], srcb[slot],
                             sems[slot][0])
        b = pltpu.async_copy(dst_hbm.at[pl.ds(base, EK)], dstb[slot],
                             sems[slot][1])
        return a, b

    def process(slot):
        def grp_body(gi, carry2):
            off = gi * GRP
            srcv = srcb[slot][pl.ds(off, GRP)]
            dstv = dstb[slot][pl.ds(off, GRP)]
            vals = [plsc.load_gather(gts[c], [srcv]) for c in range(CPS)]
            cnt, _ = plsc.scan_count(dstv)
            rlo = jnp.min(cnt)
            rhi = jnp.max(cnt)

            def round_body(r, carry3):
                m = cnt == r
                for c in range(CPS):
                    cur = plsc.load_gather(acs[c], [dstv])
                    plsc.store_scatter(acs[c], [dstv],
                                       jnp.maximum(cur, vals[c]), mask=m)
                return carry3

            lax.fori_loop(rlo, rhi + 1, round_body, 0)
            return carry2

        lax.fori_loop(0, EK // GRP, grp_body, 0)

    def wait_slot(slot):
        pltpu.make_async_copy(src_hbm.at[pl.ds(0, EK)], srcb[slot],
                              sems[slot][0]).wait()
        pltpu.make_async_copy(dst_hbm.at[pl.ds(0, EK)], dstb[slot],
                              sems[slot][1]).wait()

    fetch(0, 0)

    def pair_body(i, carry):
        ci = i * 2
        wait_slot(0)
        fetch(ci + 1, 1)
        process(0)
        wait_slot(1)

        @pl.when(ci + 2 < NCHUNK)
        def _():
            fetch(ci + 2, 0)

        process(1)
        return carry

    lax.fori_loop(0, NCHUNK // 2, pair_body, 0)

    for k in range(CPS):
        pltpu.sync_copy(acs[k], out_hbm.at[c0 + k])


def kernel(x, edge_index, W_init0, b_init0, W_init1, b_init1, W_pos0, b_pos0,
           W_pos1, b_pos1, W_local, b_local, W_global, b_global, W_final,
           b_final):
    xt = jnp.pad(x.T, ((0, 0), (0, NP - N)))
    src = edge_index[0]
    dst = edge_index[1]
    col = lambda b: b.reshape(-1, 1)

    g, cmat, dmat = _prologue(
        xt,
        W_init0.T, col(b_init0),
        W_init1.T, col(b_init1),
        W_pos0.T, col(b_pos0),
        W_pos1.T, col(b_pos1),
        W_local[F:].T,
        col(b_local),
        W_global.T, col(b_global),
        W_local[:F].T,
    )
    wgt = W_global.T
    wht = W_local[:F].T
    for i in range(2):
        m = _segmax(g, src, dst)
        g = _layer(m, cmat, dmat, wgt, wht)
    m = _segmax(g, src, dst)
    out_t = _final(m, dmat, wgt, W_final.T, col(b_final))
    return out_t[:, :N].T


# speedup vs baseline: 2.6665x; 1.1018x over previous
import functools

import jax
import jax.numpy as jnp
from jax import lax
from jax.experimental import pallas as pl
from jax.experimental.pallas import tpu as pltpu
from jax.experimental.pallas import tpu_sc as plsc

N = 10000
NP = 10240
F = 128
E = 320000
CPS = 4
EK = 8000
NCHUNK = E // EK
GRP = 16

BLK = 2048


def _prologue_body(x_ref, w0_ref, b0_ref, w1_ref, b1_ref, wp0_ref, bp0_ref,
                   wp1_ref, bp1_ref, wp_ref, bl_ref, wg_ref, bg_ref, wh_ref,
                   g_ref, c_ref, d_ref):
    dot = functools.partial(jnp.dot, preferred_element_type=jnp.float32)
    xb = x_ref[...]
    h0 = jnp.maximum(dot(w0_ref[...], xb) + b0_ref[...], 0.0)
    h = dot(w1_ref[...], h0) + b1_ref[...]
    t = jnp.maximum(dot(wp0_ref[...], h) + bp0_ref[...], 0.0)
    pos = dot(wp1_ref[...], t) + bp1_ref[...]
    p = dot(wp_ref[...], pos)
    c = p + bl_ref[...]
    d_ref[...] = bg_ref[...] - dot(wg_ref[...], p)
    c_ref[...] = c
    g_ref[...] = dot(wh_ref[...], h) + c


def _layer_body(m_ref, c_ref, d_ref, wg_ref, wh_ref, g_ref):
    dot = functools.partial(jnp.dot, preferred_element_type=jnp.float32)
    h = jnp.maximum(dot(wg_ref[...], m_ref[...]) + d_ref[...], 0.0)
    g_ref[...] = dot(wh_ref[...], h) + c_ref[...]


def _final_body(m_ref, d_ref, wg_ref, wf_ref, bf_ref, o_ref):
    dot = functools.partial(jnp.dot, preferred_element_type=jnp.float32)
    h = jnp.maximum(dot(wg_ref[...], m_ref[...]) + d_ref[...], 0.0)
    o_ref[...] = dot(wf_ref[...], h) + bf_ref[...]


def _col_spec():
    return pl.BlockSpec((F, BLK), lambda j: (0, j))


def _fix(shape):
    return pl.BlockSpec(shape, lambda j: (0, 0))


_GRID = (NP // BLK,)
_ACT = jax.ShapeDtypeStruct((F, NP), jnp.float32)

_prologue = pl.pallas_call(
    _prologue_body,
    grid=_GRID,
    in_specs=[
        _col_spec(),
        _fix((F, F)), _fix((F, 1)),
        _fix((F, F)), _fix((F, 1)),
        _fix((F, F)), _fix((F, 1)),
        _fix((3, F)), _fix((3, 1)),
        _fix((F, 3)),
        _fix((F, 1)),
        _fix((F, F)), _fix((F, 1)),
        _fix((F, F)),
    ],
    out_specs=[_col_spec(), _col_spec(), _col_spec()],
    out_shape=[_ACT, _ACT, _ACT],
)

_layer = pl.pallas_call(
    _layer_body,
    grid=_GRID,
    in_specs=[_col_spec(), _col_spec(), _col_spec(), _fix((F, F)), _fix((F, F))],
    out_specs=_col_spec(),
    out_shape=_ACT,
)

_final = pl.pallas_call(
    _final_body,
    grid=_GRID,
    in_specs=[_col_spec(), _col_spec(), _fix((F, F)), _fix((F, F)), _fix((F, 1))],
    out_specs=_col_spec(),
    out_shape=_ACT,
)


_MESH = plsc.VectorSubcoreMesh(core_axis_name="c", subcore_axis_name="s")


@functools.partial(
    pl.kernel,
    out_type=jax.ShapeDtypeStruct((F, NP), jnp.float32),
    mesh=_MESH,
    compiler_params=pltpu.CompilerParams(needs_layout_passes=False),
    scratch_types=(
        [pltpu.VMEM((NP,), jnp.float32) for _ in range(CPS)]
        + [pltpu.VMEM((NP,), jnp.float32) for _ in range(CPS)]
        + [pltpu.VMEM((EK,), jnp.int32) for _ in range(4)]
        + [pltpu.SemaphoreType.DMA for _ in range(4)]
    ),
)
def _segmax(g_hbm, src_hbm, dst_hbm, out_hbm,
            gt0, gt1, gt2, gt3, ac0, ac1, ac2, ac3,
            src0, src1, dst0, dst1, sem0, sem1, sem2, sem3):
    cid = lax.axis_index("c")
    sid = lax.axis_index("s")
    wid = sid * 2 + cid
    c0 = wid * CPS
    gts = (gt0, gt1, gt2, gt3)
    acs = (ac0, ac1, ac2, ac3)
    srcb = (src0, src1)
    dstb = (dst0, dst1)
    sems = ((sem0, sem1), (sem2, sem3))
    for k in range(CPS):
        pltpu.sync_copy(g_hbm.at[c0 + k], gts[k])
        pltpu.sync_copy(g_hbm.at[c0 + k], acs[k])

    def fetch(ci, slot):
        base = ci * EK
        a = pltpu.async_copy(src_hbm.at[pl.ds(base, EK)], srcb[slot],
                             sems[slot][0])
        b = pltpu.async_copy(dst_hbm.at[pl.ds(base, EK)], dstb[slot],
                             sems[slot][1])
        return a, b

    base = jnp.min(plsc.scan_count(jnp.zeros((GRP,), jnp.int32))[0])

    def process(slot):
        def grp_body(gi, carry2):
            off = gi * GRP
            srcv = srcb[slot][pl.ds(off, GRP)]
            dstv = dstb[slot][pl.ds(off, GRP)]
            vals = [plsc.load_gather(gts[c], [srcv]) for c in range(CPS)]
            cnt, _ = plsc.scan_count(dstv)
            m0 = cnt == base
            for c in range(CPS):
                cur = plsc.load_gather(acs[c], [dstv])
                plsc.store_scatter(acs[c], [dstv],
                                   jnp.maximum(cur, vals[c]), mask=m0)

            @pl.when(jnp.any(jnp.logical_not(m0)))
            def _():
                rhi = jnp.max(cnt)

                def round_body(r, carry3):
                    m = cnt == r
                    for c in range(CPS):
                        cur = plsc.load_gather(acs[c], [dstv])
                        plsc.store_scatter(acs[c], [dstv],
                                           jnp.maximum(cur, vals[c]), mask=m)
                    return carry3

                lax.fori_loop(base + 1, rhi + 1, round_body, 0)

            return carry2

        lax.fori_loop(0, EK // GRP, grp_body, 0)

    def wait_slot(slot):
        pltpu.make_async_copy(src_hbm.at[pl.ds(0, EK)], srcb[slot],
                              sems[slot][0]).wait()
        pltpu.make_async_copy(dst_hbm.at[pl.ds(0, EK)], dstb[slot],
                              sems[slot][1]).wait()

    fetch(0, 0)

    def pair_body(i, carry):
        ci = i * 2
        wait_slot(0)
        fetch(ci + 1, 1)
        process(0)
        wait_slot(1)

        @pl.when(ci + 2 < NCHUNK)
        def _():
            fetch(ci + 2, 0)

        process(1)
        return carry

    lax.fori_loop(0, NCHUNK // 2, pair_body, 0)

    for k in range(CPS):
        pltpu.sync_copy(acs[k], out_hbm.at[c0 + k])


def kernel(x, edge_index, W_init0, b_init0, W_init1, b_init1, W_pos0, b_pos0,
           W_pos1, b_pos1, W_local, b_local, W_global, b_global, W_final,
           b_final):
    xt = jnp.pad(x.T, ((0, 0), (0, NP - N)))
    src = edge_index[0]
    dst = edge_index[1]
    col = lambda b: b.reshape(-1, 1)

    g, cmat, dmat = _prologue(
        xt,
        W_init0.T, col(b_init0),
        W_init1.T, col(b_init1),
        W_pos0.T, col(b_pos0),
        W_pos1.T, col(b_pos1),
        W_local[F:].T,
        col(b_local),
        W_global.T, col(b_global),
        W_local[:F].T,
    )
    wgt = W_global.T
    wht = W_local[:F].T
    for i in range(2):
        m = _segmax(g, src, dst)
        g = _layer(m, cmat, dmat, wgt, wht)
    m = _segmax(g, src, dst)
    out_t = _final(m, dmat, wgt, W_final.T, col(b_final))
    return out_t[:, :N].T


# speedup vs baseline: 4.1908x; 1.5717x over previous
import functools

import jax
import jax.numpy as jnp
from jax import lax
from jax.experimental import pallas as pl
from jax.experimental.pallas import tpu as pltpu
from jax.experimental.pallas import tpu_sc as plsc

N = 10000
NP = 10240
F = 128
E = 320000
CPS = 4
EK = 8000
NCHUNK = E // EK
GRP = 16

BLK = 2048


def _prologue_body(x_ref, w0_ref, b0_ref, w1_ref, b1_ref, wp0_ref, bp0_ref,
                   wp1_ref, bp1_ref, wp_ref, bl_ref, wg_ref, bg_ref, wh_ref,
                   g_ref, c_ref, d_ref):
    dot = functools.partial(jnp.dot, preferred_element_type=jnp.float32)
    xb = x_ref[...]
    h0 = jnp.maximum(dot(w0_ref[...], xb) + b0_ref[...], 0.0)
    h = dot(w1_ref[...], h0) + b1_ref[...]
    t = jnp.maximum(dot(wp0_ref[...], h) + bp0_ref[...], 0.0)
    pos = dot(wp1_ref[...], t) + bp1_ref[...]
    p = dot(wp_ref[...], pos)
    c = p + bl_ref[...]
    d_ref[...] = bg_ref[...] - dot(wg_ref[...], p)
    c_ref[...] = c
    g_ref[...] = dot(wh_ref[...], h) + c


def _layer_body(m_ref, c_ref, d_ref, wg_ref, wh_ref, g_ref):
    dot = functools.partial(jnp.dot, preferred_element_type=jnp.float32)
    h = jnp.maximum(dot(wg_ref[...], m_ref[...]) + d_ref[...], 0.0)
    g_ref[...] = dot(wh_ref[...], h) + c_ref[...]


def _final_body(m_ref, d_ref, wg_ref, wf_ref, bf_ref, o_ref):
    dot = functools.partial(jnp.dot, preferred_element_type=jnp.float32)
    h = jnp.maximum(dot(wg_ref[...], m_ref[...]) + d_ref[...], 0.0)
    o_ref[...] = dot(wf_ref[...], h) + bf_ref[...]


def _col_spec():
    return pl.BlockSpec((F, BLK), lambda j: (0, j))


def _fix(shape):
    return pl.BlockSpec(shape, lambda j: (0, 0))


_GRID = (NP // BLK,)
_ACT = jax.ShapeDtypeStruct((F, NP), jnp.float32)

_prologue = pl.pallas_call(
    _prologue_body,
    grid=_GRID,
    in_specs=[
        _col_spec(),
        _fix((F, F)), _fix((F, 1)),
        _fix((F, F)), _fix((F, 1)),
        _fix((F, F)), _fix((F, 1)),
        _fix((3, F)), _fix((3, 1)),
        _fix((F, 3)),
        _fix((F, 1)),
        _fix((F, F)), _fix((F, 1)),
        _fix((F, F)),
    ],
    out_specs=[_col_spec(), _col_spec(), _col_spec()],
    out_shape=[_ACT, _ACT, _ACT],
)

_layer = pl.pallas_call(
    _layer_body,
    grid=_GRID,
    in_specs=[_col_spec(), _col_spec(), _col_spec(), _fix((F, F)), _fix((F, F))],
    out_specs=_col_spec(),
    out_shape=_ACT,
)

_final = pl.pallas_call(
    _final_body,
    grid=_GRID,
    in_specs=[_col_spec(), _col_spec(), _fix((F, F)), _fix((F, F)), _fix((F, 1))],
    out_specs=_col_spec(),
    out_shape=_ACT,
)


_MESH = plsc.VectorSubcoreMesh(core_axis_name="c", subcore_axis_name="s")


@functools.partial(
    pl.kernel,
    out_type=jax.ShapeDtypeStruct((F, NP), jnp.float32),
    mesh=_MESH,
    compiler_params=pltpu.CompilerParams(needs_layout_passes=False),
    scratch_types=(
        [pltpu.VMEM((NP,), jnp.float32) for _ in range(CPS)]
        + [pltpu.VMEM((NP,), jnp.float32) for _ in range(CPS)]
        + [pltpu.VMEM((EK,), jnp.int32) for _ in range(4)]
        + [pltpu.SemaphoreType.DMA for _ in range(4)]
    ),
)
def _segmax(g_hbm, src_hbm, dst_hbm, out_hbm,
            gt0, gt1, gt2, gt3, ac0, ac1, ac2, ac3,
            src0, src1, dst0, dst1, sem0, sem1, sem2, sem3):
    cid = lax.axis_index("c")
    sid = lax.axis_index("s")
    wid = sid * 2 + cid
    c0 = wid * CPS
    gts = (gt0, gt1, gt2, gt3)
    acs = (ac0, ac1, ac2, ac3)
    srcb = (src0, src1)
    dstb = (dst0, dst1)
    sems = ((sem0, sem1), (sem2, sem3))
    for k in range(CPS):
        pltpu.sync_copy(g_hbm.at[c0 + k], gts[k])
        pltpu.sync_copy(g_hbm.at[c0 + k], acs[k])

    def fetch(ci, slot):
        base = ci * EK
        a = pltpu.async_copy(src_hbm.at[pl.ds(base, EK)], srcb[slot],
                             sems[slot][0])
        b = pltpu.async_copy(dst_hbm.at[pl.ds(base, EK)], dstb[slot],
                             sems[slot][1])
        return a, b

    base = jnp.min(plsc.scan_count(jnp.zeros((GRP,), jnp.int32))[0])

    def rmw_round(vals, dstv, m):
        curs = [plsc.load_gather(acs[c], [dstv]) for c in range(CPS)]
        news = [jnp.maximum(curs[c], vals[c]) for c in range(CPS)]
        for c in range(CPS):
            plsc.store_scatter(acs[c], [dstv], news[c], mask=m)

    def process(slot):
        def grp_body(gi, rem_acc):
            off = gi * GRP
            srcv = srcb[slot][pl.ds(off, GRP)]
            dstv = dstb[slot][pl.ds(off, GRP)]
            vals = [plsc.load_gather(gts[c], [srcv]) for c in range(CPS)]
            _, last1 = plsc.scan_count(dstv)
            rest = jnp.logical_not(last1)
            _, last2 = plsc.scan_count(dstv, mask=rest)
            rmw_round(vals, dstv, last1)
            rmw_round(vals, dstv, jnp.logical_and(last2, rest))
            rem = jnp.logical_and(rest, jnp.logical_not(last2))
            return jnp.logical_or(rem_acc, rem)

        rem_acc = lax.fori_loop(0, EK // GRP, grp_body,
                                jnp.zeros((GRP,), jnp.bool_))

        @pl.when(jnp.any(rem_acc))
        def _():
            def grp_slow(gi, carry2):
                off = gi * GRP
                srcv = srcb[slot][pl.ds(off, GRP)]
                dstv = dstb[slot][pl.ds(off, GRP)]
                vals = [plsc.load_gather(gts[c], [srcv]) for c in range(CPS)]
                cnt, _ = plsc.scan_count(dstv)
                rhi = jnp.max(cnt)

                def round_body(r, carry3):
                    rmw_round(vals, dstv, cnt == r)
                    return carry3

                lax.fori_loop(base, rhi + 1, round_body, 0)
                return carry2

            lax.fori_loop(0, EK // GRP, grp_slow, 0)

    def wait_slot(slot):
        pltpu.make_async_copy(src_hbm.at[pl.ds(0, EK)], srcb[slot],
                              sems[slot][0]).wait()
        pltpu.make_async_copy(dst_hbm.at[pl.ds(0, EK)], dstb[slot],
                              sems[slot][1]).wait()

    fetch(0, 0)

    def pair_body(i, carry):
        ci = i * 2
        wait_slot(0)
        fetch(ci + 1, 1)
        process(0)
        wait_slot(1)

        @pl.when(ci + 2 < NCHUNK)
        def _():
            fetch(ci + 2, 0)

        process(1)
        return carry

    lax.fori_loop(0, NCHUNK // 2, pair_body, 0)

    for k in range(CPS):
        pltpu.sync_copy(acs[k], out_hbm.at[c0 + k])


def kernel(x, edge_index, W_init0, b_init0, W_init1, b_init1, W_pos0, b_pos0,
           W_pos1, b_pos1, W_local, b_local, W_global, b_global, W_final,
           b_final):
    xt = jnp.pad(x.T, ((0, 0), (0, NP - N)))
    src = edge_index[0]
    dst = edge_index[1]
    col = lambda b: b.reshape(-1, 1)

    g, cmat, dmat = _prologue(
        xt,
        W_init0.T, col(b_init0),
        W_init1.T, col(b_init1),
        W_pos0.T, col(b_pos0),
        W_pos1.T, col(b_pos1),
        W_local[F:].T,
        col(b_local),
        W_global.T, col(b_global),
        W_local[:F].T,
    )
    wgt = W_global.T
    wht = W_local[:F].T
    for i in range(2):
        m = _segmax(g, src, dst)
        g = _layer(m, cmat, dmat, wgt, wht)
    m = _segmax(g, src, dst)
    out_t = _final(m, dmat, wgt, W_final.T, col(b_final))
    return out_t[:, :N].T


# speedup vs baseline: 4.6543x; 1.1106x over previous
import functools

import jax
import jax.numpy as jnp
from jax import lax
from jax.experimental import pallas as pl
from jax.experimental.pallas import tpu as pltpu
from jax.experimental.pallas import tpu_sc as plsc

N = 10000
NP = 10240
F = 128
E = 320000
CPS = 4
EK = 8000
NCHUNK = E // EK
GRP = 16

BLK = 2048


def _prologue_body(x_ref, w0_ref, b0_ref, w1_ref, b1_ref, wp0_ref, bp0_ref,
                   wp1_ref, bp1_ref, wp_ref, bl_ref, wg_ref, bg_ref, wh_ref,
                   g_ref, c_ref, d_ref):
    dot = functools.partial(jnp.dot, preferred_element_type=jnp.float32)
    xb = x_ref[...]
    h0 = jnp.maximum(dot(w0_ref[...], xb) + b0_ref[...], 0.0)
    h = dot(w1_ref[...], h0) + b1_ref[...]
    t = jnp.maximum(dot(wp0_ref[...], h) + bp0_ref[...], 0.0)
    pos = dot(wp1_ref[...], t) + bp1_ref[...]
    p = dot(wp_ref[...], pos)
    c = p + bl_ref[...]
    d_ref[...] = bg_ref[...] - dot(wg_ref[...], p)
    c_ref[...] = c
    g_ref[...] = dot(wh_ref[...], h) + c


def _layer_body(m_ref, c_ref, d_ref, wg_ref, wh_ref, g_ref):
    dot = functools.partial(jnp.dot, preferred_element_type=jnp.float32)
    h = jnp.maximum(dot(wg_ref[...], m_ref[...]) + d_ref[...], 0.0)
    g_ref[...] = dot(wh_ref[...], h) + c_ref[...]


def _final_body(m_ref, d_ref, wg_ref, wf_ref, bf_ref, o_ref):
    dot = functools.partial(jnp.dot, preferred_element_type=jnp.float32)
    h = jnp.maximum(dot(wg_ref[...], m_ref[...]) + d_ref[...], 0.0)
    o_ref[...] = dot(wf_ref[...], h) + bf_ref[...]


def _col_spec():
    return pl.BlockSpec((F, BLK), lambda j: (0, j))


def _fix(shape):
    return pl.BlockSpec(shape, lambda j: (0, 0))


_GRID = (NP // BLK,)
_ACT = jax.ShapeDtypeStruct((F, NP), jnp.float32)

_prologue = pl.pallas_call(
    _prologue_body,
    grid=_GRID,
    in_specs=[
        _col_spec(),
        _fix((F, F)), _fix((F, 1)),
        _fix((F, F)), _fix((F, 1)),
        _fix((F, F)), _fix((F, 1)),
        _fix((3, F)), _fix((3, 1)),
        _fix((F, 3)),
        _fix((F, 1)),
        _fix((F, F)), _fix((F, 1)),
        _fix((F, F)),
    ],
    out_specs=[_col_spec(), _col_spec(), _col_spec()],
    out_shape=[_ACT, _ACT, _ACT],
)

_layer = pl.pallas_call(
    _layer_body,
    grid=_GRID,
    in_specs=[_col_spec(), _col_spec(), _col_spec(), _fix((F, F)), _fix((F, F))],
    out_specs=_col_spec(),
    out_shape=_ACT,
)

_final = pl.pallas_call(
    _final_body,
    grid=_GRID,
    in_specs=[_col_spec(), _col_spec(), _fix((F, F)), _fix((F, F)), _fix((F, 1))],
    out_specs=_col_spec(),
    out_shape=_ACT,
)


_MESH = plsc.VectorSubcoreMesh(core_axis_name="c", subcore_axis_name="s")


@functools.partial(
    pl.kernel,
    out_type=jax.ShapeDtypeStruct((F, NP), jnp.float32),
    mesh=_MESH,
    compiler_params=pltpu.CompilerParams(needs_layout_passes=False),
    scratch_types=(
        [pltpu.VMEM((NP,), jnp.float32) for _ in range(CPS)]
        + [pltpu.VMEM((NP,), jnp.float32) for _ in range(CPS)]
        + [pltpu.VMEM((EK,), jnp.int32) for _ in range(4)]
        + [pltpu.SemaphoreType.DMA for _ in range(4)]
    ),
)
def _segmax(g_hbm, src_hbm, dst_hbm, out_hbm,
            gt0, gt1, gt2, gt3, ac0, ac1, ac2, ac3,
            src0, src1, dst0, dst1, sem0, sem1, sem2, sem3):
    cid = lax.axis_index("c")
    sid = lax.axis_index("s")
    wid = sid * 2 + cid
    c0 = wid * CPS
    gts = (gt0, gt1, gt2, gt3)
    acs = (ac0, ac1, ac2, ac3)
    srcb = (src0, src1)
    dstb = (dst0, dst1)
    sems = ((sem0, sem1), (sem2, sem3))
    for k in range(CPS):
        pltpu.sync_copy(g_hbm.at[c0 + k], gts[k])
        pltpu.sync_copy(g_hbm.at[c0 + k], acs[k])

    def fetch(ci, slot):
        base = ci * EK
        a = pltpu.async_copy(src_hbm.at[pl.ds(base, EK)], srcb[slot],
                             sems[slot][0])
        b = pltpu.async_copy(dst_hbm.at[pl.ds(base, EK)], dstb[slot],
                             sems[slot][1])
        return a, b

    base = jnp.min(plsc.scan_count(jnp.zeros((GRP,), jnp.int32))[0])

    def rmw_round(vals, dstv, m):
        curs = [plsc.load_gather(acs[c], [dstv]) for c in range(CPS)]
        news = [jnp.maximum(curs[c], vals[c]) for c in range(CPS)]
        for c in range(CPS):
            plsc.store_scatter(acs[c], [dstv], news[c], mask=m)

    def process(slot):
        def grp_body(gi, rem_acc):
            off = gi * GRP
            srcv = srcb[slot][pl.ds(off, GRP)]
            dstv = dstb[slot][pl.ds(off, GRP)]
            vals = [plsc.load_gather(gts[c], [srcv]) for c in range(CPS)]
            cnt, last = plsc.scan_count(dstv)
            first = cnt == base
            notlast = jnp.logical_not(last)
            rmw_round(vals, dstv, last)
            rmw_round(vals, dstv, jnp.logical_and(first, notlast))
            rem = jnp.logical_and(notlast, jnp.logical_not(first))
            return jnp.logical_or(rem_acc, rem)

        rem_acc = lax.fori_loop(0, EK // GRP, grp_body,
                                jnp.zeros((GRP,), jnp.bool_), unroll=2)

        @pl.when(jnp.any(rem_acc))
        def _():
            def grp_slow(gi, carry2):
                off = gi * GRP
                srcv = srcb[slot][pl.ds(off, GRP)]
                dstv = dstb[slot][pl.ds(off, GRP)]
                vals = [plsc.load_gather(gts[c], [srcv]) for c in range(CPS)]
                cnt, _ = plsc.scan_count(dstv)
                rhi = jnp.max(cnt)

                def round_body(r, carry3):
                    rmw_round(vals, dstv, cnt == r)
                    return carry3

                lax.fori_loop(base, rhi + 1, round_body, 0)
                return carry2

            lax.fori_loop(0, EK // GRP, grp_slow, 0)

    def wait_slot(slot):
        pltpu.make_async_copy(src_hbm.at[pl.ds(0, EK)], srcb[slot],
                              sems[slot][0]).wait()
        pltpu.make_async_copy(dst_hbm.at[pl.ds(0, EK)], dstb[slot],
                              sems[slot][1]).wait()

    fetch(0, 0)

    def pair_body(i, carry):
        ci = i * 2
        wait_slot(0)
        fetch(ci + 1, 1)
        process(0)
        wait_slot(1)

        @pl.when(ci + 2 < NCHUNK)
        def _():
            fetch(ci + 2, 0)

        process(1)
        return carry

    lax.fori_loop(0, NCHUNK // 2, pair_body, 0)

    for k in range(CPS):
        pltpu.sync_copy(acs[k], out_hbm.at[c0 + k])


def kernel(x, edge_index, W_init0, b_init0, W_init1, b_init1, W_pos0, b_pos0,
           W_pos1, b_pos1, W_local, b_local, W_global, b_global, W_final,
           b_final):
    xt = jnp.pad(x.T, ((0, 0), (0, NP - N)))
    src = edge_index[0]
    dst = edge_index[1]
    col = lambda b: b.reshape(-1, 1)

    g, cmat, dmat = _prologue(
        xt,
        W_init0.T, col(b_init0),
        W_init1.T, col(b_init1),
        W_pos0.T, col(b_pos0),
        W_pos1.T, col(b_pos1),
        W_local[F:].T,
        col(b_local),
        W_global.T, col(b_global),
        W_local[:F].T,
    )
    wgt = W_global.T
    wht = W_local[:F].T
    for i in range(2):
        m = _segmax(g, src, dst)
        g = _layer(m, cmat, dmat, wgt, wht)
    m = _segmax(g, src, dst)
    out_t = _final(m, dmat, wgt, W_final.T, col(b_final))
    return out_t[:, :N].T
